# baseline (speedup 1.0000x reference)
import functools

import jax
import jax.numpy as jnp
import numpy as np
from jax import lax
from jax.experimental import pallas as pl
from jax.experimental.pallas import tpu as pltpu
from jax.experimental.pallas import tpu_sc as plsc

D = 64
K = 128
SB = 25
NB = 5
ZR = 125
NPASS = 4
NSUB = 16


def _make_spmm(R, E, C):
    assert R % (2 * NPASS) == 0 and E % K == 0
    NCH = E // K
    assert NCH % SB == 0 and SB % NB == 0
    NSUPER = NCH // SB
    Rq = R // (2 * NPASS)
    assert Rq % ZR == 0
    NZB = Rq // ZR
    NZI = -(-NZB // NSUB)
    MAXI = -(-NSUPER // NSUB)
    NG = K // 16

    mesh = plsc.VectorSubcoreMesh(core_axis_name="c", subcore_axis_name="s",
                                  num_cores=2, num_subcores=NSUB)

    @functools.partial(
        pl.kernel,
        out_type=jax.ShapeDtypeStruct((R, D), jnp.float32),
        mesh=mesh,
        compiler_params=pltpu.CompilerParams(use_tc_tiling_on_sc=False, needs_layout_passes=False),
        scratch_types=[
            pltpu.VMEM_SHARED((Rq, D), jnp.float32),
            [pltpu.VMEM((K, D // 2), jnp.int32)] * NB,
            [pltpu.VMEM((K, D), jnp.float32)] * NB,
            pltpu.VMEM((SB, 3, K), jnp.int32),
            pltpu.VMEM((ZR, D), jnp.float32),
            pltpu.VMEM((16,), jnp.int32),
            [pltpu.SemaphoreType.DMA] * NB,
            [pltpu.SemaphoreType.DMA] * NB,
        ],
    )
    def k(x_hbm, pack_hbm, bounds_hbm, out_hbm,
          accum, gbufs, sbufs, packv, zbuf, partv, gsems, ssems):
        cid = lax.axis_index("c")
        sid = lax.axis_index("s")

        def zb(i, carry):
            for d4 in range(D // 16):
                zbuf[i, pl.ds(d4 * 16, 16)] = jnp.zeros((16,), jnp.float32)
            return carry
        lax.fori_loop(0, ZR, zb, 0)

        pltpu.sync_copy(bounds_hbm, partv)
        pv = partv[pl.ds(0, 16)]

        hi_mask = jnp.full((16,), -65536, jnp.int32)

        def scale(jm, gbuf, sbuf):
            def sg(g, carry):
                e0 = g * 16
                v16 = plsc.bitcast(packv[jm, 1, pl.ds(e0, 16)], jnp.float32)
                for j in range(16):
                    v = v16[j]
                    for blk in range(D // 32):
                        w = gbuf[e0 + j, pl.ds(blk * 16, 16)]
                        fe = plsc.bitcast(lax.shift_left(w, 16), jnp.float32)
                        fo = plsc.bitcast(w & hi_mask, jnp.float32)
                        sbuf[e0 + j, pl.ds(blk * 32, 16)] = fe * v
                        sbuf[e0 + j, pl.ds(blk * 32 + 16, 16)] = fo * v
                return carry
            lax.fori_loop(0, NG, sg, 0)

        def super_body(u, e_lo, e_hi):
            base0 = pl.multiple_of(u * SB, SB)
            pltpu.sync_copy(pack_hbm.at[pl.ds(base0, SB)], packv)
            for b in range(NB):
                pltpu.async_copy(x_hbm.at[packv.at[b, 0]], gbufs[b], gsems[b])

            ebase = base0 * K
            full = (ebase >= e_lo) & (ebase + SB * K <= e_hi)

            @pl.when(jnp.logical_not(full))
            def _():
                def mk(g, carry):
                    jm = lax.div(g, NG)
                    sl = pl.ds(lax.rem(g, NG) * 16, 16)
                    gidx = ebase + g * 16 + lax.iota(jnp.int32, 16)
                    m = (gidx >= e_lo) & (gidx < e_hi)
                    vf = plsc.bitcast(packv[jm, 1, sl], jnp.float32)
                    vf = jnp.where(m, vf, jnp.zeros((16,), jnp.float32))
                    packv[jm, 1, sl] = plsc.bitcast(vf, jnp.int32)
                    return carry
                lax.fori_loop(0, SB * NG, mk, 0)

            def ring(m, carry):
                for b in range(NB):
                    jm = NB * m + b
                    pltpu.make_async_copy(x_hbm.at[packv.at[jm, 0]],
                                          gbufs[b], gsems[b]).wait()

                    @pl.when(m > 0)
                    def _():
                        pltpu.make_async_copy(
                            sbufs[b], accum.at[packv.at[jm - NB, 2]],
                            ssems[b]).wait()
                    scale(jm, gbufs[b], sbufs[b])
                    pltpu.async_copy(sbufs[b], accum.at[packv.at[jm, 2]],
                                     ssems[b], add=True)

                    @pl.when(jm + NB < SB)
                    def _():
                        pltpu.async_copy(x_hbm.at[packv.at[jm + NB, 0]],
                                         gbufs[b], gsems[b])
                return carry

            lax.fori_loop(0, SB // NB, ring, 0)
            for b in range(NB):
                pltpu.make_async_copy(sbufs[b],
                                      accum.at[packv.at[SB - NB + b, 2]],
                                      ssems[b]).wait()

        def sel(qq, lo):
            r = pv[0 + lo]
            for z in range(1, 2 * NPASS):
                r = jnp.where(qq == z, pv[z + lo], r)
            return r

        for p in range(NPASS):
            q = NPASS * cid + p
            e_lo = sel(q, 0)
            e_hi = sel(q, 1)
            row_base = q * Rq
            u_lo = lax.div(e_lo, K * SB)
            u_hi = lax.div(e_hi + (K * SB - 1), K * SB)
            u_hi = jnp.where(e_lo == e_hi, u_lo, u_hi)

            for i in range(NZI):
                b = sid + i * NSUB

                @pl.when(b < NZB)
                def _():
                    pltpu.sync_copy(zbuf, accum.at[pl.ds(b * ZR, ZR), :])
            plsc.subcore_barrier()

            def it(i, carry):
                u = u_lo + sid + i * NSUB

                @pl.when(u < u_hi)
                def _():
                    super_body(u, e_lo, e_hi)
                return carry

            lax.fori_loop(0, MAXI, it, 0)
            plsc.subcore_barrier()

            for i in range(NZI):
                b = sid + i * NSUB

                @pl.when(b < NZB)
                def _():
                    pltpu.sync_copy(accum.at[pl.ds(b * ZR, ZR), :],
                                    out_hbm.at[pl.ds(row_base + b * ZR, ZR), :])
            plsc.subcore_barrier()

    return k


def _prelu(x, a):
    return jnp.where(x >= 0, x, a * x)


def _stage_a(s0, w, a):
    A = s0.shape[0]
    BLK = 2000
    grid = (A // BLK,)

    def body(s_ref, w_ref, a_ref, emb_ref, ego_ref):
        av = a_ref[0]
        e = _prelu(s_ref[...], av)
        emb_ref[...] = e
        ego_ref[...] = jnp.dot(
            e, w_ref[...], preferred_element_type=jnp.float32
        ).astype(jnp.bfloat16)

    return pl.pallas_call(
        body,
        grid=grid,
        in_specs=[
            pl.BlockSpec((BLK, D), lambda i: (i, 0)),
            pl.BlockSpec((D, D), lambda i: (0, 0)),
            pl.BlockSpec(memory_space=pltpu.SMEM),
        ],
        out_specs=[
            pl.BlockSpec((BLK, D), lambda i: (i, 0)),
            pl.BlockSpec((BLK, D), lambda i: (i, 0)),
        ],
        out_shape=[
            jax.ShapeDtypeStruct((A, D), jnp.float32),
            jax.ShapeDtypeStruct((A, D), jnp.bfloat16),
        ],
    )(s0, w, a)


def _stage_b(emb0_3, s1_3, W, b, a):
    Uu = emb0_3.shape[0]
    UB = 2000
    grid = (Uu // UB,)

    def body(e0_ref, s1_ref, w_ref, b_ref, a_ref, u_ref):
        av = a_ref[0]
        h0 = jnp.sum(e0_ref[...], axis=1)
        h1 = jnp.sum(_prelu(s1_ref[...], av), axis=1)
        social = jnp.concatenate([h0, h1], axis=1) * 0.25
        u = lax.dot_general(social, w_ref[...], (((1,), (0,)), ((), ())),
                            preferred_element_type=jnp.float32)
        u_ref[...] = (u + b_ref[...][None, :]).astype(jnp.bfloat16)

    return pl.pallas_call(
        body,
        grid=grid,
        in_specs=[
            pl.BlockSpec((UB, 5, D), lambda i: (i, 0, 0)),
            pl.BlockSpec((UB, 5, D), lambda i: (i, 0, 0)),
            pl.BlockSpec((2 * D, D), lambda i: (0, 0)),
            pl.BlockSpec((D,), lambda i: (0,)),
            pl.BlockSpec(memory_space=pltpu.SMEM),
        ],
        out_specs=pl.BlockSpec((UB, D), lambda i: (i, 0)),
        out_shape=jax.ShapeDtypeStruct((Uu, D), jnp.bfloat16),
    )(emb0_3, s1_3, W, b, a)


def _stage_c(s2, wu, wi, a, n_user_blocks):
    N = s2.shape[0]
    BLK = 2000
    grid = (N // BLK,)

    def body(s_ref, wu_ref, wi_ref, a_ref, emb_ref, ego_ref):
        av = a_ref[0]
        pid = pl.program_id(0)
        e = _prelu(s_ref[...], av)
        emb_ref[...] = e
        w = jnp.where(pid < n_user_blocks, wu_ref[...], wi_ref[...])
        ego_ref[...] = jnp.dot(
            e, w, preferred_element_type=jnp.float32).astype(jnp.bfloat16)

    return pl.pallas_call(
        body,
        grid=grid,
        in_specs=[
            pl.BlockSpec((BLK, D), lambda i: (i, 0)),
            pl.BlockSpec((D, D), lambda i: (0, 0)),
            pl.BlockSpec((D, D), lambda i: (0, 0)),
            pl.BlockSpec(memory_space=pltpu.SMEM),
        ],
        out_specs=[
            pl.BlockSpec((BLK, D), lambda i: (i, 0)),
            pl.BlockSpec((BLK, D), lambda i: (i, 0)),
        ],
        out_shape=[
            jax.ShapeDtypeStruct((N, D), jnp.float32),
            jax.ShapeDtypeStruct((N, D), jnp.bfloat16),
        ],
    )(s2, wu, wi, a)


def _stage_d_user(emb_u, s3_u, a):
    Uu = emb_u.shape[0]
    UB = 2000
    grid = (Uu // UB,)

    def body(e_ref, s_ref, a_ref, o_ref):
        av = a_ref[0]
        o_ref[...] = jnp.concatenate([e_ref[...], _prelu(s_ref[...], av)], axis=1)

    return pl.pallas_call(
        body,
        grid=grid,
        in_specs=[
            pl.BlockSpec((UB, D), lambda i: (i, 0)),
            pl.BlockSpec((UB, D), lambda i: (i, 0)),
            pl.BlockSpec(memory_space=pltpu.SMEM),
        ],
        out_specs=pl.BlockSpec((UB, 2 * D), lambda i: (i, 0)),
        out_shape=jax.ShapeDtypeStruct((Uu, 2 * D), jnp.float32),
    )(emb_u, s3_u, a)


def _stage_d_item(emb_i3, s3_i3, a):
    Ii = emb_i3.shape[0]
    IB = 1000
    grid = (Ii // IB,)

    def body(e_ref, s_ref, a_ref, o_ref):
        av = a_ref[0]
        a0 = jnp.sum(e_ref[...], axis=1)
        a1 = jnp.sum(_prelu(s_ref[...], av), axis=1)
        o_ref[...] = jnp.concatenate([a0, a1], axis=1) * 0.2

    return pl.pallas_call(
        body,
        grid=grid,
        in_specs=[
            pl.BlockSpec((IB, 5, D), lambda i: (i, 0, 0)),
            pl.BlockSpec((IB, 5, D), lambda i: (i, 0, 0)),
            pl.BlockSpec(memory_space=pltpu.SMEM),
        ],
        out_specs=pl.BlockSpec((IB, 2 * D), lambda i: (i, 0)),
        out_shape=jax.ShapeDtypeStruct((Ii, 2 * D), jnp.float32),
    )(emb_i3, s3_i3, a)


def kernel(user_table, item_table, social_vals, adj_vals, w_r1_W, w_r1_b,
           prelu_a, social_w1, user_w1, item_w1,
           social_rows, social_cols, adj_rows, adj_cols):
    A = user_table.shape[0]
    Uu = A // 5
    Ii = item_table.shape[0] // 5
    N = Uu + item_table.shape[0]
    ES = social_rows.shape[0]
    EA = adj_rows.shape[0]

    s_rows = social_rows.astype(jnp.int32)
    a_rows = adj_rows.astype(jnp.int32)

    def _pack(cols, vals, rows, R, E):
        rows_m = lax.rem(rows, jnp.int32(R // (2 * NPASS)))
        vi = lax.bitcast_convert_type(vals, jnp.int32)
        p = jnp.stack([cols.astype(jnp.int32).reshape(E // K, K),
                       vi.reshape(E // K, K),
                       rows_m.reshape(E // K, K)], axis=1)
        return p

    s_pack = _pack(social_cols, social_vals, s_rows, A, ES)
    a_pack = _pack(adj_cols, adj_vals, a_rows, N, EA)

    def _bounds(rows, R, E):
        nq = 2 * NPASS
        qs = jnp.arange(nq + 1, dtype=jnp.int32) * (R // nq)
        b = jnp.searchsorted(rows, qs).astype(jnp.int32)
        b = b.at[0].set(0).at[nq].set(E)
        return jnp.concatenate([b, jnp.zeros((16 - nq - 1,), jnp.int32)])

    s_split = _bounds(s_rows, A, ES)
    a_split = _bounds(a_rows, N, EA)

    spmm_s = _make_spmm(A, ES, A)
    spmm_a = _make_spmm(N, EA, N)

    j = np.arange(D)
    perm = np.asarray(32 * (j // 32) + 16 * (j % 2) + (j % 32) // 2)
    pmat = jnp.asarray(np.eye(D, dtype=np.float32)[perm].T)

    def _to_pairs(x_bf):
        return lax.bitcast_convert_type(
            x_bf.reshape(x_bf.shape[0], D // 2, 2), jnp.int32)

    ut_pack = _to_pairs(user_table[:, perm].astype(jnp.bfloat16))
    it_bf = item_table[:, perm].astype(jnp.bfloat16)

    s0 = spmm_s(ut_pack, s_pack, s_split)
    emb0, ego = _stage_a(s0, social_w1 @ pmat, prelu_a)
    s1 = spmm_s(_to_pairs(ego), s_pack, s_split)
    u = _stage_b(emb0.reshape(Uu, 5, D), s1.reshape(Uu, 5, D),
                 w_r1_W.T @ pmat, w_r1_b @ pmat, prelu_a)
    ego0 = jnp.concatenate([u, it_bf], axis=0)
    s2 = spmm_a(_to_pairs(ego0), a_pack, a_split)
    emb, ego1 = _stage_c(s2, user_w1 @ pmat, item_w1 @ pmat, prelu_a,
                         Uu // 2000)
    s3 = spmm_a(_to_pairs(ego1), a_pack, a_split)
    user_embedding = _stage_d_user(emb[:Uu], s3[:Uu], prelu_a)
    item_embedding = _stage_d_item(emb[Uu:].reshape(Ii, 5, D),
                                   s3[Uu:].reshape(Ii, 5, D), prelu_a)
    return (user_embedding, item_embedding)


# speedup vs baseline: 1.9009x; 1.9009x over previous
import functools

import jax
import jax.numpy as jnp
import numpy as np
from jax import lax
from jax.experimental import pallas as pl
from jax.experimental.pallas import tpu as pltpu
from jax.experimental.pallas import tpu_sc as plsc

D = 64
K = 128
SB = 25
NB = 5
ZR = 125
NPASS = 4
NSUB = 16


def _make_spmm(R, E, C):
    assert R % (2 * NPASS) == 0 and E % K == 0
    NCH = E // K
    assert NCH % SB == 0 and SB % NB == 0
    NSUPER = NCH // SB
    Rq = R // (2 * NPASS)
    assert Rq % ZR == 0
    NZB = Rq // ZR
    NZI = -(-NZB // NSUB)
    MAXI = -(-NSUPER // NSUB)
    NG = K // 16

    mesh = plsc.VectorSubcoreMesh(core_axis_name="c", subcore_axis_name="s",
                                  num_cores=2, num_subcores=NSUB)

    @functools.partial(
        pl.kernel,
        out_type=jax.ShapeDtypeStruct((R, D), jnp.float32),
        mesh=mesh,
        compiler_params=pltpu.CompilerParams(use_tc_tiling_on_sc=False, needs_layout_passes=False),
        scratch_types=[
            pltpu.VMEM_SHARED((Rq, D), jnp.float32),
            [pltpu.VMEM((K, D), jnp.float32)] * NB,
            [pltpu.VMEM((K, D), jnp.float32)] * NB,
            pltpu.VMEM((SB, 3, K), jnp.int32),
            pltpu.VMEM((ZR, D), jnp.float32),
            pltpu.VMEM((16,), jnp.int32),
            [pltpu.SemaphoreType.DMA] * NB,
            [pltpu.SemaphoreType.DMA] * NB,
        ],
    )
    def k(x_hbm, pack_hbm, bounds_hbm, out_hbm,
          accum, gbufs, sbufs, packv, zbuf, partv, gsems, ssems):
        cid = lax.axis_index("c")
        sid = lax.axis_index("s")

        def zb(i, carry):
            for d4 in range(D // 16):
                zbuf[i, pl.ds(d4 * 16, 16)] = jnp.zeros((16,), jnp.float32)
            return carry
        lax.fori_loop(0, ZR, zb, 0)

        pltpu.sync_copy(bounds_hbm, partv)
        pv = partv[pl.ds(0, 16)]

        def scale(jm, gbuf, sbuf):
            def sg(g, carry):
                e0 = g * 16
                v16 = plsc.bitcast(packv[jm, 1, pl.ds(e0, 16)], jnp.float32)
                for j in range(16):
                    v = v16[j]
                    for d4 in range(D // 16):
                        sl = pl.ds(d4 * 16, 16)
                        sbuf[e0 + j, sl] = gbuf[e0 + j, sl] * v
                return carry
            lax.fori_loop(0, NG, sg, 0)

        def super_body(u, e_lo, e_hi):
            base0 = pl.multiple_of(u * SB, SB)
            pltpu.sync_copy(pack_hbm.at[pl.ds(base0, SB)], packv)
            for b in range(NB):
                pltpu.async_copy(x_hbm.at[packv.at[b, 0]], gbufs[b], gsems[b])

            ebase = base0 * K
            full = (ebase >= e_lo) & (ebase + SB * K <= e_hi)

            @pl.when(jnp.logical_not(full))
            def _():
                def mk(g, carry):
                    jm = lax.div(g, NG)
                    sl = pl.ds(lax.rem(g, NG) * 16, 16)
                    gidx = ebase + g * 16 + lax.iota(jnp.int32, 16)
                    m = (gidx >= e_lo) & (gidx < e_hi)
                    vf = plsc.bitcast(packv[jm, 1, sl], jnp.float32)
                    vf = jnp.where(m, vf, jnp.zeros((16,), jnp.float32))
                    packv[jm, 1, sl] = plsc.bitcast(vf, jnp.int32)
                    return carry
                lax.fori_loop(0, SB * NG, mk, 0)

            def ring(m, carry):
                for b in range(NB):
                    jm = NB * m + b
                    pltpu.make_async_copy(x_hbm.at[packv.at[jm, 0]],
                                          gbufs[b], gsems[b]).wait()

                    @pl.when(m > 0)
                    def _():
                        pltpu.make_async_copy(
                            sbufs[b], accum.at[packv.at[jm - NB, 2]],
                            ssems[b]).wait()
                    scale(jm, gbufs[b], sbufs[b])
                    pltpu.async_copy(sbufs[b], accum.at[packv.at[jm, 2]],
                                     ssems[b], add=True)

                    @pl.when(jm + NB < SB)
                    def _():
                        pltpu.async_copy(x_hbm.at[packv.at[jm + NB, 0]],
                                         gbufs[b], gsems[b])
                return carry

            lax.fori_loop(0, SB // NB, ring, 0)
            for b in range(NB):
                pltpu.make_async_copy(sbufs[b],
                                      accum.at[packv.at[SB - NB + b, 2]],
                                      ssems[b]).wait()

        def sel(qq, lo):
            r = pv[0 + lo]
            for z in range(1, 2 * NPASS):
                r = jnp.where(qq == z, pv[z + lo], r)
            return r

        for p in range(NPASS):
            q = NPASS * cid + p
            e_lo = sel(q, 0)
            e_hi = sel(q, 1)
            row_base = q * Rq
            u_lo = lax.div(e_lo, K * SB)
            u_hi = lax.div(e_hi + (K * SB - 1), K * SB)
            u_hi = jnp.where(e_lo == e_hi, u_lo, u_hi)

            for i in range(NZI):
                b = sid + i * NSUB

                @pl.when(b < NZB)
                def _():
                    pltpu.sync_copy(zbuf, accum.at[pl.ds(b * ZR, ZR), :])
            plsc.subcore_barrier()

            def it(i, carry):
                u = u_lo + sid + i * NSUB

                @pl.when(u < u_hi)
                def _():
                    super_body(u, e_lo, e_hi)
                return carry

            lax.fori_loop(0, MAXI, it, 0)
            plsc.subcore_barrier()

            for i in range(NZI):
                b = sid + i * NSUB

                @pl.when(b < NZB)
                def _():
                    pltpu.sync_copy(accum.at[pl.ds(b * ZR, ZR), :],
                                    out_hbm.at[pl.ds(row_base + b * ZR, ZR), :])
            plsc.subcore_barrier()

    return k


def _prelu(x, a):
    return jnp.where(x >= 0, x, a * x)


def _stage_a(s0, w, a):
    A = s0.shape[0]
    BLK = 2000
    grid = (A // BLK,)

    def body(s_ref, w_ref, a_ref, emb_ref, ego_ref):
        av = a_ref[0]
        e = _prelu(s_ref[...], av)
        emb_ref[...] = e
        ego_ref[...] = jnp.dot(e, w_ref[...],
                               preferred_element_type=jnp.float32)

    return pl.pallas_call(
        body,
        grid=grid,
        in_specs=[
            pl.BlockSpec((BLK, D), lambda i: (i, 0)),
            pl.BlockSpec((D, D), lambda i: (0, 0)),
            pl.BlockSpec(memory_space=pltpu.SMEM),
        ],
        out_specs=[
            pl.BlockSpec((BLK, D), lambda i: (i, 0)),
            pl.BlockSpec((BLK, D), lambda i: (i, 0)),
        ],
        out_shape=[
            jax.ShapeDtypeStruct((A, D), jnp.float32),
            jax.ShapeDtypeStruct((A, D), jnp.float32),
        ],
    )(s0, w, a)


def _stage_b(emb0_3, s1_3, W, b, a):
    Uu = emb0_3.shape[0]
    UB = 2000
    grid = (Uu // UB,)

    def body(e0_ref, s1_ref, w_ref, b_ref, a_ref, u_ref):
        av = a_ref[0]
        h0 = jnp.sum(e0_ref[...], axis=1)
        h1 = jnp.sum(_prelu(s1_ref[...], av), axis=1)
        social = jnp.concatenate([h0, h1], axis=1) * 0.25
        u = lax.dot_general(social, w_ref[...], (((1,), (0,)), ((), ())),
                            preferred_element_type=jnp.float32)
        u_ref[...] = u + b_ref[...][None, :]

    return pl.pallas_call(
        body,
        grid=grid,
        in_specs=[
            pl.BlockSpec((UB, 5, D), lambda i: (i, 0, 0)),
            pl.BlockSpec((UB, 5, D), lambda i: (i, 0, 0)),
            pl.BlockSpec((2 * D, D), lambda i: (0, 0)),
            pl.BlockSpec((D,), lambda i: (0,)),
            pl.BlockSpec(memory_space=pltpu.SMEM),
        ],
        out_specs=pl.BlockSpec((UB, D), lambda i: (i, 0)),
        out_shape=jax.ShapeDtypeStruct((Uu, D), jnp.float32),
    )(emb0_3, s1_3, W, b, a)


def _stage_c(s2, wu, wi, a, n_user_blocks):
    N = s2.shape[0]
    BLK = 2000
    grid = (N // BLK,)

    def body(s_ref, wu_ref, wi_ref, a_ref, emb_ref, ego_ref):
        av = a_ref[0]
        pid = pl.program_id(0)
        e = _prelu(s_ref[...], av)
        emb_ref[...] = e
        w = jnp.where(pid < n_user_blocks, wu_ref[...], wi_ref[...])
        ego_ref[...] = jnp.dot(e, w, preferred_element_type=jnp.float32)

    return pl.pallas_call(
        body,
        grid=grid,
        in_specs=[
            pl.BlockSpec((BLK, D), lambda i: (i, 0)),
            pl.BlockSpec((D, D), lambda i: (0, 0)),
            pl.BlockSpec((D, D), lambda i: (0, 0)),
            pl.BlockSpec(memory_space=pltpu.SMEM),
        ],
        out_specs=[
            pl.BlockSpec((BLK, D), lambda i: (i, 0)),
            pl.BlockSpec((BLK, D), lambda i: (i, 0)),
        ],
        out_shape=[
            jax.ShapeDtypeStruct((N, D), jnp.float32),
            jax.ShapeDtypeStruct((N, D), jnp.float32),
        ],
    )(s2, wu, wi, a)


def _stage_d_user(emb_u, s3_u, a):
    Uu = emb_u.shape[0]
    UB = 2000
    grid = (Uu // UB,)

    def body(e_ref, s_ref, a_ref, o_ref):
        av = a_ref[0]
        o_ref[...] = jnp.concatenate([e_ref[...], _prelu(s_ref[...], av)], axis=1)

    return pl.pallas_call(
        body,
        grid=grid,
        in_specs=[
            pl.BlockSpec((UB, D), lambda i: (i, 0)),
            pl.BlockSpec((UB, D), lambda i: (i, 0)),
            pl.BlockSpec(memory_space=pltpu.SMEM),
        ],
        out_specs=pl.BlockSpec((UB, 2 * D), lambda i: (i, 0)),
        out_shape=jax.ShapeDtypeStruct((Uu, 2 * D), jnp.float32),
    )(emb_u, s3_u, a)


def _stage_d_item(emb_i3, s3_i3, a):
    Ii = emb_i3.shape[0]
    IB = 1000
    grid = (Ii // IB,)

    def body(e_ref, s_ref, a_ref, o_ref):
        av = a_ref[0]
        a0 = jnp.sum(e_ref[...], axis=1)
        a1 = jnp.sum(_prelu(s_ref[...], av), axis=1)
        o_ref[...] = jnp.concatenate([a0, a1], axis=1) * 0.2

    return pl.pallas_call(
        body,
        grid=grid,
        in_specs=[
            pl.BlockSpec((IB, 5, D), lambda i: (i, 0, 0)),
            pl.BlockSpec((IB, 5, D), lambda i: (i, 0, 0)),
            pl.BlockSpec(memory_space=pltpu.SMEM),
        ],
        out_specs=pl.BlockSpec((IB, 2 * D), lambda i: (i, 0)),
        out_shape=jax.ShapeDtypeStruct((Ii, 2 * D), jnp.float32),
    )(emb_i3, s3_i3, a)


def kernel(user_table, item_table, social_vals, adj_vals, w_r1_W, w_r1_b,
           prelu_a, social_w1, user_w1, item_w1,
           social_rows, social_cols, adj_rows, adj_cols):
    A = user_table.shape[0]
    Uu = A // 5
    Ii = item_table.shape[0] // 5
    N = Uu + item_table.shape[0]
    ES = social_rows.shape[0]
    EA = adj_rows.shape[0]

    s_rows = social_rows.astype(jnp.int32)
    a_rows = adj_rows.astype(jnp.int32)

    def _pack(cols, vals, rows, R, E):
        rows_m = lax.rem(rows, jnp.int32(R // (2 * NPASS)))
        vi = lax.bitcast_convert_type(vals, jnp.int32)
        p = jnp.stack([cols.astype(jnp.int32).reshape(E // K, K),
                       vi.reshape(E // K, K),
                       rows_m.reshape(E // K, K)], axis=1)
        return p

    s_pack = _pack(social_cols, social_vals, s_rows, A, ES)
    a_pack = _pack(adj_cols, adj_vals, a_rows, N, EA)

    def _bounds(rows, R, E):
        nq = 2 * NPASS
        qs = jnp.arange(nq + 1, dtype=jnp.int32) * (R // nq)
        b = jnp.searchsorted(rows, qs).astype(jnp.int32)
        b = b.at[0].set(0).at[nq].set(E)
        return jnp.concatenate([b, jnp.zeros((16 - nq - 1,), jnp.int32)])

    s_split = _bounds(s_rows, A, ES)
    a_split = _bounds(a_rows, N, EA)

    spmm_s = _make_spmm(A, ES, A)
    spmm_a = _make_spmm(N, EA, N)

    s0 = spmm_s(user_table, s_pack, s_split)
    emb0, ego = _stage_a(s0, social_w1, prelu_a)
    s1 = spmm_s(ego, s_pack, s_split)
    u = _stage_b(emb0.reshape(Uu, 5, D), s1.reshape(Uu, 5, D),
                 w_r1_W.T, w_r1_b, prelu_a)
    ego0 = jnp.concatenate([u, item_table], axis=0)
    s2 = spmm_a(ego0, a_pack, a_split)
    emb, ego1 = _stage_c(s2, user_w1, item_w1, prelu_a, Uu // 2000)
    s3 = spmm_a(ego1, a_pack, a_split)
    user_embedding = _stage_d_user(emb[:Uu], s3[:Uu], prelu_a)
    item_embedding = _stage_d_item(emb[Uu:].reshape(Ii, 5, D),
                                   s3[Uu:].reshape(Ii, 5, D), prelu_a)
    return (user_embedding, item_embedding)


# speedup vs baseline: 1.9146x; 1.0072x over previous
import functools

import jax
import jax.numpy as jnp
import numpy as np
from jax import lax
from jax.experimental import pallas as pl
from jax.experimental.pallas import tpu as pltpu
from jax.experimental.pallas import tpu_sc as plsc

D = 64
K = 128
SB = 25
NB = 5
ZR = 125
NPASS = 4
NSUB = 16


def _make_spmm(R, E, C):
    assert R % (2 * NPASS) == 0 and E % K == 0
    NCH = E // K
    assert NCH % SB == 0 and SB % NB == 0
    NSUPER = NCH // SB
    Rq = R // (2 * NPASS)
    assert Rq % ZR == 0
    NZB = Rq // ZR
    NZI = -(-NZB // NSUB)
    MAXI = -(-NSUPER // NSUB)
    NG = K // 16

    mesh = plsc.VectorSubcoreMesh(core_axis_name="c", subcore_axis_name="s",
                                  num_cores=2, num_subcores=NSUB)

    @functools.partial(
        pl.kernel,
        out_type=jax.ShapeDtypeStruct((R, D), jnp.float32),
        mesh=mesh,
        compiler_params=pltpu.CompilerParams(use_tc_tiling_on_sc=False, needs_layout_passes=False),
        scratch_types=[
            pltpu.VMEM_SHARED((Rq, D), jnp.float32),
            [pltpu.VMEM((K, D), jnp.float32)] * NB,
            [pltpu.VMEM((K, D), jnp.float32)] * NB,
            pltpu.VMEM((SB, 3, K), jnp.int32),
            pltpu.VMEM((ZR, D), jnp.float32),
            pltpu.VMEM((16,), jnp.int32),
            [pltpu.SemaphoreType.DMA] * NB,
            [pltpu.SemaphoreType.DMA] * NB,
        ],
    )
    def k(x_hbm, pack_hbm, bounds_hbm, out_hbm,
          accum, gbufs, sbufs, packv, zbuf, partv, gsems, ssems):
        cid = lax.axis_index("c")
        sid = lax.axis_index("s")

        def zb(i, carry):
            for d4 in range(D // 16):
                zbuf[i, pl.ds(d4 * 16, 16)] = jnp.zeros((16,), jnp.float32)
            return carry
        lax.fori_loop(0, ZR, zb, 0)

        pltpu.sync_copy(bounds_hbm, partv)
        pv = partv[pl.ds(0, 16)]

        def scale(jm, gbuf, sbuf):
            def sg(g, carry):
                e0 = g * 16
                v16 = plsc.bitcast(packv[jm, 1, pl.ds(e0, 16)], jnp.float32)
                for j in range(16):
                    v = v16[j]
                    for d4 in range(D // 16):
                        sl = pl.ds(d4 * 16, 16)
                        sbuf[e0 + j, sl] = gbuf[e0 + j, sl] * v
                return carry
            lax.fori_loop(0, NG, sg, 0)

        def super_body(u, e_lo, e_hi):
            base0 = pl.multiple_of(u * SB, SB)
            pltpu.sync_copy(pack_hbm.at[pl.ds(base0, SB)], packv)
            for b in range(NB):
                pltpu.async_copy(x_hbm.at[packv.at[b, 0]], gbufs[b], gsems[b])

            ebase = base0 * K
            full = (ebase >= e_lo) & (ebase + SB * K <= e_hi)

            @pl.when(jnp.logical_not(full))
            def _():
                def mk(g, carry):
                    jm = lax.div(g, NG)
                    sl = pl.ds(lax.rem(g, NG) * 16, 16)
                    gidx = ebase + g * 16 + lax.iota(jnp.int32, 16)
                    m = (gidx >= e_lo) & (gidx < e_hi)
                    vf = plsc.bitcast(packv[jm, 1, sl], jnp.float32)
                    vf = jnp.where(m, vf, jnp.zeros((16,), jnp.float32))
                    packv[jm, 1, sl] = plsc.bitcast(vf, jnp.int32)
                    return carry
                lax.fori_loop(0, SB * NG, mk, 0)

            def ring(m, carry):
                for b in range(NB):
                    jm = NB * m + b
                    pltpu.make_async_copy(x_hbm.at[packv.at[jm, 0]],
                                          gbufs[b], gsems[b]).wait()

                    @pl.when(m > 0)
                    def _():
                        pltpu.make_async_copy(
                            sbufs[b], accum.at[packv.at[jm - NB, 2]],
                            ssems[b]).wait()
                    scale(jm, gbufs[b], sbufs[b])
                    pltpu.async_copy(sbufs[b], accum.at[packv.at[jm, 2]],
                                     ssems[b], add=True)

                    @pl.when(jm + NB < SB)
                    def _():
                        pltpu.async_copy(x_hbm.at[packv.at[jm + NB, 0]],
                                         gbufs[b], gsems[b])
                return carry

            lax.fori_loop(0, SB // NB, ring, 0)
            for b in range(NB):
                pltpu.make_async_copy(sbufs[b],
                                      accum.at[packv.at[SB - NB + b, 2]],
                                      ssems[b]).wait()

        def sel(qq, lo):
            r = pv[0 + lo]
            for z in range(1, 2 * NPASS):
                r = jnp.where(qq == z, pv[z + lo], r)
            return r

        for p in range(NPASS):
            q = NPASS * cid + p
            e_lo = sel(q, 0)
            e_hi = sel(q, 1)
            row_base = q * Rq
            u_lo = lax.div(e_lo, K * SB)
            u_hi = lax.div(e_hi + (K * SB - 1), K * SB)
            u_hi = jnp.where(e_lo == e_hi, u_lo, u_hi)

            for i in range(NZI):
                b = sid + i * NSUB

                @pl.when(b < NZB)
                def _():
                    pltpu.async_copy(zbuf, accum.at[pl.ds(b * ZR, ZR), :],
                                     gsems[0])
            for i in range(NZI):
                b = sid + i * NSUB

                @pl.when(b < NZB)
                def _():
                    pltpu.make_async_copy(
                        zbuf, accum.at[pl.ds(b * ZR, ZR), :], gsems[0]).wait()
            plsc.subcore_barrier()

            def it(i, carry):
                u = u_lo + sid + i * NSUB

                @pl.when(u < u_hi)
                def _():
                    super_body(u, e_lo, e_hi)
                return carry

            lax.fori_loop(0, MAXI, it, 0)
            plsc.subcore_barrier()

            for i in range(NZI):
                b = sid + i * NSUB

                @pl.when(b < NZB)
                def _():
                    pltpu.async_copy(
                        accum.at[pl.ds(b * ZR, ZR), :],
                        out_hbm.at[pl.ds(row_base + b * ZR, ZR), :], gsems[0])
            for i in range(NZI):
                b = sid + i * NSUB

                @pl.when(b < NZB)
                def _():
                    pltpu.make_async_copy(
                        accum.at[pl.ds(b * ZR, ZR), :],
                        out_hbm.at[pl.ds(row_base + b * ZR, ZR), :],
                        gsems[0]).wait()
            plsc.subcore_barrier()

    return k


def _prelu(x, a):
    return jnp.where(x >= 0, x, a * x)


def _stage_a(s0, w, a):
    A = s0.shape[0]
    BLK = 2000
    grid = (A // BLK,)

    def body(s_ref, w_ref, a_ref, emb_ref, ego_ref):
        av = a_ref[0]
        e = _prelu(s_ref[...], av)
        emb_ref[...] = e
        ego_ref[...] = jnp.dot(e, w_ref[...],
                               preferred_element_type=jnp.float32)

    return pl.pallas_call(
        body,
        grid=grid,
        in_specs=[
            pl.BlockSpec((BLK, D), lambda i: (i, 0)),
            pl.BlockSpec((D, D), lambda i: (0, 0)),
            pl.BlockSpec(memory_space=pltpu.SMEM),
        ],
        out_specs=[
            pl.BlockSpec((BLK, D), lambda i: (i, 0)),
            pl.BlockSpec((BLK, D), lambda i: (i, 0)),
        ],
        out_shape=[
            jax.ShapeDtypeStruct((A, D), jnp.float32),
            jax.ShapeDtypeStruct((A, D), jnp.float32),
        ],
    )(s0, w, a)


def _stage_b(emb0_3, s1_3, W, b, a):
    Uu = emb0_3.shape[0]
    UB = 2000
    grid = (Uu // UB,)

    def body(e0_ref, s1_ref, w_ref, b_ref, a_ref, u_ref):
        av = a_ref[0]
        h0 = jnp.sum(e0_ref[...], axis=1)
        h1 = jnp.sum(_prelu(s1_ref[...], av), axis=1)
        social = jnp.concatenate([h0, h1], axis=1) * 0.25
        u = lax.dot_general(social, w_ref[...], (((1,), (0,)), ((), ())),
                            preferred_element_type=jnp.float32)
        u_ref[...] = u + b_ref[...][None, :]

    return pl.pallas_call(
        body,
        grid=grid,
        in_specs=[
            pl.BlockSpec((UB, 5, D), lambda i: (i, 0, 0)),
            pl.BlockSpec((UB, 5, D), lambda i: (i, 0, 0)),
            pl.BlockSpec((2 * D, D), lambda i: (0, 0)),
            pl.BlockSpec((D,), lambda i: (0,)),
            pl.BlockSpec(memory_space=pltpu.SMEM),
        ],
        out_specs=pl.BlockSpec((UB, D), lambda i: (i, 0)),
        out_shape=jax.ShapeDtypeStruct((Uu, D), jnp.float32),
    )(emb0_3, s1_3, W, b, a)


def _stage_c(s2, wu, wi, a, n_user_blocks):
    N = s2.shape[0]
    BLK = 2000
    grid = (N // BLK,)

    def body(s_ref, wu_ref, wi_ref, a_ref, emb_ref, ego_ref):
        av = a_ref[0]
        pid = pl.program_id(0)
        e = _prelu(s_ref[...], av)
        emb_ref[...] = e
        w = jnp.where(pid < n_user_blocks, wu_ref[...], wi_ref[...])
        ego_ref[...] = jnp.dot(e, w, preferred_element_type=jnp.float32)

    return pl.pallas_call(
        body,
        grid=grid,
        in_specs=[
            pl.BlockSpec((BLK, D), lambda i: (i, 0)),
            pl.BlockSpec((D, D), lambda i: (0, 0)),
            pl.BlockSpec((D, D), lambda i: (0, 0)),
            pl.BlockSpec(memory_space=pltpu.SMEM),
        ],
        out_specs=[
            pl.BlockSpec((BLK, D), lambda i: (i, 0)),
            pl.BlockSpec((BLK, D), lambda i: (i, 0)),
        ],
        out_shape=[
            jax.ShapeDtypeStruct((N, D), jnp.float32),
            jax.ShapeDtypeStruct((N, D), jnp.float32),
        ],
    )(s2, wu, wi, a)


def _stage_d_user(emb_u, s3_u, a):
    Uu = emb_u.shape[0]
    UB = 2000
    grid = (Uu // UB,)

    def body(e_ref, s_ref, a_ref, o_ref):
        av = a_ref[0]
        o_ref[...] = jnp.concatenate([e_ref[...], _prelu(s_ref[...], av)], axis=1)

    return pl.pallas_call(
        body,
        grid=grid,
        in_specs=[
            pl.BlockSpec((UB, D), lambda i: (i, 0)),
            pl.BlockSpec((UB, D), lambda i: (i, 0)),
            pl.BlockSpec(memory_space=pltpu.SMEM),
        ],
        out_specs=pl.BlockSpec((UB, 2 * D), lambda i: (i, 0)),
        out_shape=jax.ShapeDtypeStruct((Uu, 2 * D), jnp.float32),
    )(emb_u, s3_u, a)


def _stage_d_item(emb_i3, s3_i3, a):
    Ii = emb_i3.shape[0]
    IB = 1000
    grid = (Ii // IB,)

    def body(e_ref, s_ref, a_ref, o_ref):
        av = a_ref[0]
        a0 = jnp.sum(e_ref[...], axis=1)
        a1 = jnp.sum(_prelu(s_ref[...], av), axis=1)
        o_ref[...] = jnp.concatenate([a0, a1], axis=1) * 0.2

    return pl.pallas_call(
        body,
        grid=grid,
        in_specs=[
            pl.BlockSpec((IB, 5, D), lambda i: (i, 0, 0)),
            pl.BlockSpec((IB, 5, D), lambda i: (i, 0, 0)),
            pl.BlockSpec(memory_space=pltpu.SMEM),
        ],
        out_specs=pl.BlockSpec((IB, 2 * D), lambda i: (i, 0)),
        out_shape=jax.ShapeDtypeStruct((Ii, 2 * D), jnp.float32),
    )(emb_i3, s3_i3, a)


def kernel(user_table, item_table, social_vals, adj_vals, w_r1_W, w_r1_b,
           prelu_a, social_w1, user_w1, item_w1,
           social_rows, social_cols, adj_rows, adj_cols):
    A = user_table.shape[0]
    Uu = A // 5
    Ii = item_table.shape[0] // 5
    N = Uu + item_table.shape[0]
    ES = social_rows.shape[0]
    EA = adj_rows.shape[0]

    s_rows = social_rows.astype(jnp.int32)
    a_rows = adj_rows.astype(jnp.int32)

    def _pack(cols, vals, rows, R, E):
        rows_m = lax.rem(rows, jnp.int32(R // (2 * NPASS)))
        vi = lax.bitcast_convert_type(vals, jnp.int32)
        p = jnp.stack([cols.astype(jnp.int32).reshape(E // K, K),
                       vi.reshape(E // K, K),
                       rows_m.reshape(E // K, K)], axis=1)
        return p

    s_pack = _pack(social_cols, social_vals, s_rows, A, ES)
    a_pack = _pack(adj_cols, adj_vals, a_rows, N, EA)

    def _bounds(rows, R, E):
        nq = 2 * NPASS
        qs = jnp.arange(nq + 1, dtype=jnp.int32) * (R // nq)
        b = jnp.searchsorted(rows, qs).astype(jnp.int32)
        b = b.at[0].set(0).at[nq].set(E)
        return jnp.concatenate([b, jnp.zeros((16 - nq - 1,), jnp.int32)])

    s_split = _bounds(s_rows, A, ES)
    a_split = _bounds(a_rows, N, EA)

    spmm_s = _make_spmm(A, ES, A)
    spmm_a = _make_spmm(N, EA, N)

    s0 = spmm_s(user_table, s_pack, s_split)
    emb0, ego = _stage_a(s0, social_w1, prelu_a)
    s1 = spmm_s(ego, s_pack, s_split)
    u = _stage_b(emb0.reshape(Uu, 5, D), s1.reshape(Uu, 5, D),
                 w_r1_W.T, w_r1_b, prelu_a)
    ego0 = jnp.concatenate([u, item_table], axis=0)
    s2 = spmm_a(ego0, a_pack, a_split)
    emb, ego1 = _stage_c(s2, user_w1, item_w1, prelu_a, Uu // 2000)
    s3 = spmm_a(ego1, a_pack, a_split)
    user_embedding = _stage_d_user(emb[:Uu], s3[:Uu], prelu_a)
    item_embedding = _stage_d_item(emb[Uu:].reshape(Ii, 5, D),
                                   s3[Uu:].reshape(Ii, 5, D), prelu_a)
    return (user_embedding, item_embedding)
